# baseline (speedup 1.0000x reference)
import functools

import jax
import jax.numpy as jnp
from jax import lax
from jax.experimental import pallas as pl
from jax.experimental.pallas import tpu as pltpu
from jax.experimental.pallas import tpu_sc as plsc

_N = 10000
_E = 320000
_D = 128
_HE = 10000
_HID = 128
_DH = 64

_NT = 16
_L = 16

_RPT = 160
_RTOT = _NT * _RPT
_EPAD = _RTOT * 128
_NP = 10240
_SL = 640
_IB = 16
_NG = _RPT // 4
_PAD_ID = 10000


def _sc_propagate(xh, nid2, eidp, z2, z1):
  mesh = plsc.VectorSubcoreMesh(core_axis_name="c", subcore_axis_name="s")

  @functools.partial(
      pl.kernel,
      out_type=[jax.ShapeDtypeStruct((2, _N, _DH), jnp.float32),
                jax.ShapeDtypeStruct((2 * _NP, _DH), jnp.float32)],
      mesh=mesh,
      scratch_types=[
          pltpu.VMEM((512, _DH), jnp.float32),
          pltpu.VMEM((2, _IB, 128), jnp.int32),
          pltpu.VMEM((2, _IB, 128), jnp.int32),
          pltpu.VMEM((128,), jnp.float32),
          pltpu.VMEM((_SL,), jnp.float32),
          pltpu.VMEM((_SL + _L,), jnp.float32),
          pltpu.VMEM_SHARED((_NP, _DH), jnp.float32),
          pltpu.VMEM_SHARED((_NP, _DH), jnp.float32),
          pltpu.VMEM_SHARED((_NP,), jnp.float32),
          pltpu.VMEM_SHARED((_NP,), jnp.float32),
          pltpu.SemaphoreType.DMA,
          pltpu.SemaphoreType.DMA,
          pltpu.SemaphoreType.DMA,
          pltpu.SemaphoreType.DMA,
          pltpu.SemaphoreType.DMA,
          pltpu.SemaphoreType.DMA,
          pltpu.SemaphoreType.DMA,
          pltpu.SemaphoreType.DMA,
      ],
      compiler_params=pltpu.CompilerParams(use_tc_tiling_on_sc=False),
  )
  def k(xh_hbm, nid_hbm, eid_hbm, z2_hbm, z1_hbm, out_hbm, msg_hbm,
        buf, gidx, sidx, ones_v, vals_v, inv_v,
        acc, oacc, edeg, deg, *sems):
    gsems = sems[0:4]
    ssems = sems[4:8]
    c = lax.axis_index("c")
    s = lax.axis_index("s")
    base = s * _SL

    def drain_rows(sm):
      pltpu.make_async_copy(z2_hbm.at[pl.ds(0, 128)],
                            buf.at[pl.ds(0, 128)], sm).wait()

    def drain_ones(sm):
      pltpu.make_async_copy(z1_hbm.at[pl.ds(0, 128)], ones_v, sm).wait()

    for j in range(128 // _L):
      ones_v[pl.ds(j * _L, _L)] = jnp.full((_L,), 1.0, jnp.float32)

    pltpu.sync_copy(z2_hbm, acc.at[pl.ds(base, _SL)])
    pltpu.sync_copy(z2_hbm, oacc.at[pl.ds(base, _SL)])
    pltpu.sync_copy(z1_hbm, edeg.at[pl.ds(base, _SL)])
    pltpu.sync_copy(z1_hbm, deg.at[pl.ds(base, _SL)])

    plsc.subcore_barrier()

    def run_pass(gtable, load_blk, sdst, cdst):
      def fire_gather(h, t):
        bslot = lax.rem(h // _IB, 2)
        r = lax.rem(h, _IB)
        pltpu.async_copy(gtable.at[gidx.at[bslot, r]],
                         buf.at[pl.ds(t * 128, 128)], gsems[t])

      def fire_scatter(h, t):
        bslot = lax.rem(h // _IB, 2)
        r = lax.rem(h, _IB)
        pltpu.async_copy(buf.at[pl.ds(t * 128, 128)],
                         sdst.at[sidx.at[bslot, r]], ssems[t], add=True)
        pltpu.async_copy(ones_v, cdst.at[sidx.at[bslot, r]], ssems[t],
                         add=True)

      load_blk(0, 0)

      def body(i, carry):
        @pl.when((lax.rem(i, 4) == 0) & (i > 0))
        def _():
          b = i // 4
          load_blk(b, lax.rem(b, 2))

        for t in range(4):
          h = 4 * i + t

          @pl.when(i >= 1)
          def _():
            drain_rows(ssems[t])
            drain_ones(ssems[t])
          fire_gather(h, t)
          t2 = (t + 2) % 4

          @pl.when((i >= 1) | (t >= 2))
          def _():
            drain_rows(gsems[t2])
            fire_scatter(h - 2, t2)
        return carry

      lax.fori_loop(0, _NG, body, 0)
      ns = 4 * _NG
      for h in (ns - 2, ns - 1):
        t = h % 4
        drain_rows(gsems[t])
        fire_scatter(h, t)
      for t in range(4):
        drain_rows(ssems[t])
        drain_ones(ssems[t])

    def scale_slice(src, off, write_back):
      pltpu.sync_copy(src.at[pl.ds(base + off, 320)], buf.at[pl.ds(0, 320)])

      def scale_body(r, carry):
        sp = jnp.zeros((_L,), jnp.float32) + inv_v[pl.ds(r + off, _L)][0]
        for q in range(_DH // _L):
          buf[r, pl.ds(q * _L, _L)] = buf[r, pl.ds(q * _L, _L)] * sp
        return carry

      lax.fori_loop(0, 320, scale_body, 0)
      if write_back:
        pltpu.sync_copy(buf.at[pl.ds(0, 320)],
                        msg_hbm.at[pl.ds(c * _NP + base + off, 320)])

    def load_recips(darr):
      pltpu.sync_copy(darr.at[pl.ds(base, _SL)], vals_v)
      for j in range(_SL // _L):
        v = vals_v[pl.ds(j * _L, _L)]
        inv_v[pl.ds(j * _L, _L)] = jnp.where(v > 0.0, 1.0 / v, 0.0)

    def load_blk1(b, slot):
      rb = s * _RPT + b * _IB
      pltpu.sync_copy(nid_hbm.at[c, pl.ds(rb, _IB)], gidx.at[slot])
      pltpu.sync_copy(eid_hbm.at[0, pl.ds(rb, _IB)], sidx.at[slot])

    run_pass(xh_hbm, load_blk1, acc, edeg)
    plsc.subcore_barrier()

    load_recips(edeg)
    scale_slice(acc, 0, True)
    scale_slice(acc, 320, True)
    plsc.subcore_barrier()

    def load_blk2(b, slot):
      rb = s * _RPT + b * _IB
      pltpu.sync_copy(eid_hbm.at[c, pl.ds(rb, _IB)], gidx.at[slot])
      pltpu.sync_copy(nid_hbm.at[0, pl.ds(rb, _IB)], sidx.at[slot])

    run_pass(msg_hbm, load_blk2, oacc, deg)
    plsc.subcore_barrier()

    load_recips(deg)
    for off in (0, 320):
      scale_slice(oacc, off, False)

      @pl.when(s < _NT - 1)
      def _():
        pltpu.sync_copy(buf.at[pl.ds(0, 320)],
                        out_hbm.at[c, pl.ds(base + off, 320)])

      last0 = (_NT - 1) * _SL + off
      sz15 = min(320, _N - last0)
      @pl.when(s == _NT - 1)
      def _():
        pltpu.sync_copy(buf.at[pl.ds(0, sz15)],
                        out_hbm.at[c, pl.ds(last0, sz15)])

  return k(xh, nid2, eidp, z2, z1)


_RB = 2000


def _stage1_body(px0_ref, px1_ref, w0_ref, b0_ref, w1_ref, b1_ref,
                 a1w_ref, a1b_ref, wd_ref, bd_ref, ss_ref):
  i = pl.program_id(0)
  px0 = px0_ref[0]
  px1 = px1_ref[0]
  w0 = w0_ref[...]
  w1 = w1_ref[...]
  out0 = (jnp.dot(px0, w0[:_DH], preferred_element_type=jnp.float32)
          + jnp.dot(px1, w0[_DH:], preferred_element_type=jnp.float32)
          + b0_ref[...])
  out1 = (jnp.dot(px0, w1[:_DH], preferred_element_type=jnp.float32)
          + jnp.dot(px1, w1[_DH:], preferred_element_type=jnp.float32)
          + b1_ref[...])
  nf = (out0 + out1) * 0.5
  a1 = jnp.maximum(
      jnp.dot(nf, a1w_ref[...], preferred_element_type=jnp.float32) + a1b_ref[...],
      0.0)
  d = jnp.sum(a1 * wd_ref[...], axis=1, keepdims=True) + bd_ref[0, 0]
  att0 = 1.0 / (1.0 + jnp.exp(d))
  part = jnp.sum(att0, axis=0, keepdims=True)

  @pl.when(i == 0)
  def _():
    ss_ref[...] = jnp.zeros_like(ss_ref)

  ss_ref[...] += part


def _tc_stage1(px2, w0, b0, w1, b1, a1w, a1b, wd, bd):
  full = pl.BlockSpec((_D, _D), lambda i: (0, 0))
  vec = pl.BlockSpec((1, _D), lambda i: (0, 0))
  one = pl.BlockSpec((1, 1), lambda i: (0, 0))
  half = pl.BlockSpec((1, _RB, _DH), lambda i: (0, i, 0))
  return pl.pallas_call(
      _stage1_body,
      grid=(_N // _RB,),
      in_specs=[half, half, full, vec, full, vec, full, vec, vec, one],
      out_specs=one,
      out_shape=jax.ShapeDtypeStruct((1, 1), jnp.float32),
  )(px2[0:1], px2[1:2], w0, b0, w1, b1, a1w, a1b, wd, bd)


def _layernorm_relu(h, g_ref, be_ref):
  mu = jnp.mean(h, axis=1, keepdims=True)
  var = jnp.mean((h - mu) ** 2, axis=1, keepdims=True)
  return jnp.maximum((h - mu) * lax.rsqrt(var + 1e-5) * g_ref[...] + be_ref[...],
                     0.0)


def _stage2_body(px0_ref, px1_ref, w0_ref, b0_ref, w1_ref, b1_ref,
                 t0w_ref, t0b_ref, g0_ref, be0_ref,
                 t1w_ref, t1b_ref, g1_ref, be1_ref, c_ref, out_ref):
  px0 = px0_ref[0]
  px1 = px1_ref[0]
  w0 = w0_ref[...]
  w1 = w1_ref[...]
  out0 = (jnp.dot(px0, w0[:_DH], preferred_element_type=jnp.float32)
          + jnp.dot(px1, w0[_DH:], preferred_element_type=jnp.float32)
          + b0_ref[...])
  out1 = (jnp.dot(px0, w1[:_DH], preferred_element_type=jnp.float32)
          + jnp.dot(px1, w1[_DH:], preferred_element_type=jnp.float32)
          + b1_ref[...])
  h0 = jnp.dot(out0, t0w_ref[...], preferred_element_type=jnp.float32) + t0b_ref[...]
  h1 = jnp.dot(out1, t1w_ref[...], preferred_element_type=jnp.float32) + t1b_ref[...]
  t0 = _layernorm_relu(h0, g0_ref, be0_ref)
  t1 = _layernorm_relu(h1, g1_ref, be1_ref)
  out_ref[...] = c_ref[0, 0] * t0 + c_ref[0, 1] * t1


def _tc_stage2(px2, w0, b0, w1, b1, t0w, t0b, g0, be0, t1w, t1b, g1, be1, c01):
  full = pl.BlockSpec((_D, _D), lambda i: (0, 0))
  vec = pl.BlockSpec((1, _D), lambda i: (0, 0))
  two = pl.BlockSpec((1, 2), lambda i: (0, 0))
  half = pl.BlockSpec((1, _RB, _DH), lambda i: (0, i, 0))
  blk = pl.BlockSpec((_RB, _D), lambda i: (i, 0))
  return pl.pallas_call(
      _stage2_body,
      grid=(_N // _RB,),
      in_specs=[half, half, full, vec, full, vec, full, vec, vec, vec,
                full, vec, vec, vec, two],
      out_specs=blk,
      out_shape=jax.ShapeDtypeStruct((_N, _D), jnp.float32),
  )(px2[0:1], px2[1:2], w0, b0, w1, b1, t0w, t0b, g0, be0,
    t1w, t1b, g1, be1, c01)


def kernel(x, hyperedge_index, W0, b0, W1, b1, scale_weights,
           A1_w, A1_b, A2_w, A2_b, T0_w, T0_b, g0, be0, T1_w, T1_b, g1, be1):
  nidx = hyperedge_index[0].astype(jnp.int32)
  eidx = hyperedge_index[1].astype(jnp.int32)

  pad = jnp.full((_EPAD - _E,), _PAD_ID, jnp.int32)
  nid_p = jnp.concatenate([nidx, pad]).reshape(_RTOT, 128)
  nid2 = jnp.stack([nid_p, nid_p + _NP])
  eid_p = jnp.concatenate([eidx, pad]).reshape(_RTOT, 128)
  eid2 = jnp.stack([eid_p, eid_p + _NP])

  xh = jnp.pad(x.reshape(_N, 2, _DH).transpose(1, 0, 2),
               ((0, 0), (0, _NP - _N), (0, 0))).reshape(2 * _NP, _DH)
  z2 = jnp.zeros((_SL, _DH), jnp.float32)
  z1 = jnp.zeros((_SL,), jnp.float32)

  px2, _ = _sc_propagate(xh, nid2, eid2, z2, z1)

  wd = (A2_w[:, 1] - A2_w[:, 0]).reshape(1, _HID)
  bd = (A2_b[1] - A2_b[0]).reshape(1, 1)
  b0r = b0.reshape(1, _D)
  b1r = b1.reshape(1, _D)

  ssum = _tc_stage1(px2, W0, b0r, W1, b1r, A1_w, A1_b.reshape(1, _HID), wd, bd)
  dyn0 = ssum[0, 0] / _N
  sw = jax.nn.softmax(scale_weights)
  c0 = (sw[0] + dyn0) * 0.5
  c1 = (sw[1] + (1.0 - dyn0)) * 0.5
  c01 = jnp.stack([c0, c1]).reshape(1, 2)

  return _tc_stage2(px2, W0, b0r, W1, b1r,
                    T0_w, T0_b.reshape(1, _D), g0.reshape(1, _D),
                    be0.reshape(1, _D),
                    T1_w, T1_b.reshape(1, _D), g1.reshape(1, _D),
                    be1.reshape(1, _D), c01)


# speedup vs baseline: 1.6909x; 1.6909x over previous
import functools

import jax
import jax.numpy as jnp
from jax import lax
from jax.experimental import pallas as pl
from jax.experimental.pallas import tpu as pltpu
from jax.experimental.pallas import tpu_sc as plsc

_N = 10000
_E = 320000
_D = 128
_HE = 10000
_HID = 128
_DH = 64

_NT = 16
_L = 16

_RPT = 160
_RTOT = _NT * _RPT
_EPAD = _RTOT * 128
_NP = 10240
_SL = 640
_IB = 16
_NG = _RPT // 4
_PAD_ID = 10000


def _sc_propagate(xh, nidp, eidp, z2, z1):
  mesh = plsc.VectorSubcoreMesh(core_axis_name="c", subcore_axis_name="s")

  @functools.partial(
      pl.kernel,
      out_type=jax.ShapeDtypeStruct((2, _N, _DH), jnp.float32),
      mesh=mesh,
      scratch_types=[
          pltpu.VMEM((512, _DH), jnp.float32),
          pltpu.VMEM((2, _IB, 128), jnp.int32),
          pltpu.VMEM((2, _IB, 128), jnp.int32),
          pltpu.VMEM((128,), jnp.float32),
          pltpu.VMEM((_SL,), jnp.float32),
          pltpu.VMEM((_SL + _L,), jnp.float32),
          pltpu.VMEM_SHARED((_NP, _DH), jnp.float32),
          pltpu.VMEM_SHARED((_NP, _DH), jnp.float32),
          pltpu.VMEM_SHARED((_NP,), jnp.float32),
          pltpu.VMEM_SHARED((_NP,), jnp.float32),
          pltpu.SemaphoreType.DMA,
          pltpu.SemaphoreType.DMA,
          pltpu.SemaphoreType.DMA,
          pltpu.SemaphoreType.DMA,
          pltpu.SemaphoreType.DMA,
          pltpu.SemaphoreType.DMA,
          pltpu.SemaphoreType.DMA,
          pltpu.SemaphoreType.DMA,
      ],
      compiler_params=pltpu.CompilerParams(use_tc_tiling_on_sc=False),
  )
  def k(xh_hbm, nid_hbm, eid_hbm, z2_hbm, z1_hbm, out_hbm,
        buf, gidx, sidx, ones_v, vals_v, inv_v,
        acc, xs, edeg, deg, *sems):
    gsems = sems[0:4]
    ssems = sems[4:8]
    c = lax.axis_index("c")
    s = lax.axis_index("s")
    base = s * _SL

    def drain_rows(sm):
      pltpu.make_async_copy(z2_hbm.at[pl.ds(0, 128)],
                            buf.at[pl.ds(0, 128)], sm).wait()

    def drain_ones(sm):
      pltpu.make_async_copy(z1_hbm.at[pl.ds(0, 128)], ones_v, sm).wait()

    for j in range(128 // _L):
      ones_v[pl.ds(j * _L, _L)] = jnp.full((_L,), 1.0, jnp.float32)

    pltpu.sync_copy(z2_hbm, acc.at[pl.ds(base, _SL)])
    pltpu.sync_copy(xh_hbm.at[pl.ds(c * _NP + base, _SL)],
                    xs.at[pl.ds(base, _SL)])
    pltpu.sync_copy(z1_hbm, edeg.at[pl.ds(base, _SL)])
    pltpu.sync_copy(z1_hbm, deg.at[pl.ds(base, _SL)])

    plsc.subcore_barrier()

    def run_pass(gtable, load_blk, sdst, cdst):
      def fire_gather(h, t):
        bslot = lax.rem(h // _IB, 2)
        r = lax.rem(h, _IB)
        pltpu.async_copy(gtable.at[gidx.at[bslot, r]],
                         buf.at[pl.ds(t * 128, 128)], gsems[t])

      def fire_scatter(h, t):
        bslot = lax.rem(h // _IB, 2)
        r = lax.rem(h, _IB)
        pltpu.async_copy(buf.at[pl.ds(t * 128, 128)],
                         sdst.at[sidx.at[bslot, r]], ssems[t], add=True)
        pltpu.async_copy(ones_v, cdst.at[sidx.at[bslot, r]], ssems[t],
                         add=True)

      load_blk(0, 0)

      def body(i, carry):
        @pl.when((lax.rem(i, 4) == 0) & (i > 0))
        def _():
          b = i // 4
          load_blk(b, lax.rem(b, 2))

        for t in range(4):
          h = 4 * i + t

          @pl.when(i >= 1)
          def _():
            drain_rows(ssems[t])
            drain_ones(ssems[t])
          fire_gather(h, t)
          t2 = (t + 2) % 4

          @pl.when((i >= 1) | (t >= 2))
          def _():
            drain_rows(gsems[t2])
            fire_scatter(h - 2, t2)
        return carry

      lax.fori_loop(0, _NG, body, 0)
      ns = 4 * _NG
      for h in (ns - 2, ns - 1):
        t = h % 4
        drain_rows(gsems[t])
        fire_scatter(h, t)
      for t in range(4):
        drain_rows(ssems[t])
        drain_ones(ssems[t])

    def scale_slice(src, off, write_back):
      pltpu.sync_copy(src.at[pl.ds(base + off, 320)], buf.at[pl.ds(0, 320)])

      def scale_body(r, carry):
        sp = jnp.zeros((_L,), jnp.float32) + inv_v[pl.ds(r + off, _L)][0]
        for q in range(_DH // _L):
          buf[r, pl.ds(q * _L, _L)] = buf[r, pl.ds(q * _L, _L)] * sp
        return carry

      lax.fori_loop(0, 320, scale_body, 0)
      if write_back:
        pltpu.sync_copy(buf.at[pl.ds(0, 320)], src.at[pl.ds(base + off, 320)])

    def load_recips(darr):
      pltpu.sync_copy(darr.at[pl.ds(base, _SL)], vals_v)
      for j in range(_SL // _L):
        v = vals_v[pl.ds(j * _L, _L)]
        inv_v[pl.ds(j * _L, _L)] = jnp.where(v > 0.0, 1.0 / v, 0.0)

    def load_blk1(b, slot):
      rb = s * _RPT + b * _IB
      pltpu.sync_copy(nid_hbm.at[pl.ds(rb, _IB)], gidx.at[slot])
      pltpu.sync_copy(eid_hbm.at[pl.ds(rb, _IB)], sidx.at[slot])

    run_pass(xs, load_blk1, acc, edeg)
    plsc.subcore_barrier()

    load_recips(edeg)
    scale_slice(acc, 0, True)
    scale_slice(acc, 320, True)
    pltpu.sync_copy(z2_hbm, xs.at[pl.ds(base, _SL)])
    plsc.subcore_barrier()

    def load_blk2(b, slot):
      rb = s * _RPT + b * _IB
      pltpu.sync_copy(eid_hbm.at[pl.ds(rb, _IB)], gidx.at[slot])
      pltpu.sync_copy(nid_hbm.at[pl.ds(rb, _IB)], sidx.at[slot])

    run_pass(acc, load_blk2, xs, deg)
    plsc.subcore_barrier()

    load_recips(deg)
    for off in (0, 320):
      scale_slice(xs, off, False)

      @pl.when(s < _NT - 1)
      def _():
        pltpu.sync_copy(buf.at[pl.ds(0, 320)],
                        out_hbm.at[c, pl.ds(base + off, 320)])

      last0 = (_NT - 1) * _SL + off
      sz15 = min(320, _N - last0)
      @pl.when(s == _NT - 1)
      def _():
        pltpu.sync_copy(buf.at[pl.ds(0, sz15)],
                        out_hbm.at[c, pl.ds(last0, sz15)])

  return k(xh, nidp, eidp, z2, z1)


_RB = 2000


def _stage1_body(px0_ref, px1_ref, w0_ref, b0_ref, w1_ref, b1_ref,
                 a1w_ref, a1b_ref, wd_ref, bd_ref, ss_ref):
  i = pl.program_id(0)
  px0 = px0_ref[0]
  px1 = px1_ref[0]
  w0 = w0_ref[...]
  w1 = w1_ref[...]
  out0 = (jnp.dot(px0, w0[:_DH], preferred_element_type=jnp.float32)
          + jnp.dot(px1, w0[_DH:], preferred_element_type=jnp.float32)
          + b0_ref[...])
  out1 = (jnp.dot(px0, w1[:_DH], preferred_element_type=jnp.float32)
          + jnp.dot(px1, w1[_DH:], preferred_element_type=jnp.float32)
          + b1_ref[...])
  nf = (out0 + out1) * 0.5
  a1 = jnp.maximum(
      jnp.dot(nf, a1w_ref[...], preferred_element_type=jnp.float32) + a1b_ref[...],
      0.0)
  d = jnp.sum(a1 * wd_ref[...], axis=1, keepdims=True) + bd_ref[0, 0]
  att0 = 1.0 / (1.0 + jnp.exp(d))
  part = jnp.sum(att0, axis=0, keepdims=True)

  @pl.when(i == 0)
  def _():
    ss_ref[...] = jnp.zeros_like(ss_ref)

  ss_ref[...] += part


def _tc_stage1(px2, w0, b0, w1, b1, a1w, a1b, wd, bd):
  full = pl.BlockSpec((_D, _D), lambda i: (0, 0))
  vec = pl.BlockSpec((1, _D), lambda i: (0, 0))
  one = pl.BlockSpec((1, 1), lambda i: (0, 0))
  half = pl.BlockSpec((1, _RB, _DH), lambda i: (0, i, 0))
  return pl.pallas_call(
      _stage1_body,
      grid=(_N // _RB,),
      in_specs=[half, half, full, vec, full, vec, full, vec, vec, one],
      out_specs=one,
      out_shape=jax.ShapeDtypeStruct((1, 1), jnp.float32),
  )(px2[0:1], px2[1:2], w0, b0, w1, b1, a1w, a1b, wd, bd)


def _layernorm_relu(h, g_ref, be_ref):
  mu = jnp.mean(h, axis=1, keepdims=True)
  var = jnp.mean((h - mu) ** 2, axis=1, keepdims=True)
  return jnp.maximum((h - mu) * lax.rsqrt(var + 1e-5) * g_ref[...] + be_ref[...],
                     0.0)


def _stage2_body(px0_ref, px1_ref, w0_ref, b0_ref, w1_ref, b1_ref,
                 t0w_ref, t0b_ref, g0_ref, be0_ref,
                 t1w_ref, t1b_ref, g1_ref, be1_ref, c_ref, out_ref):
  px0 = px0_ref[0]
  px1 = px1_ref[0]
  w0 = w0_ref[...]
  w1 = w1_ref[...]
  out0 = (jnp.dot(px0, w0[:_DH], preferred_element_type=jnp.float32)
          + jnp.dot(px1, w0[_DH:], preferred_element_type=jnp.float32)
          + b0_ref[...])
  out1 = (jnp.dot(px0, w1[:_DH], preferred_element_type=jnp.float32)
          + jnp.dot(px1, w1[_DH:], preferred_element_type=jnp.float32)
          + b1_ref[...])
  h0 = jnp.dot(out0, t0w_ref[...], preferred_element_type=jnp.float32) + t0b_ref[...]
  h1 = jnp.dot(out1, t1w_ref[...], preferred_element_type=jnp.float32) + t1b_ref[...]
  t0 = _layernorm_relu(h0, g0_ref, be0_ref)
  t1 = _layernorm_relu(h1, g1_ref, be1_ref)
  out_ref[...] = c_ref[0, 0] * t0 + c_ref[0, 1] * t1


def _tc_stage2(px2, w0, b0, w1, b1, t0w, t0b, g0, be0, t1w, t1b, g1, be1, c01):
  full = pl.BlockSpec((_D, _D), lambda i: (0, 0))
  vec = pl.BlockSpec((1, _D), lambda i: (0, 0))
  two = pl.BlockSpec((1, 2), lambda i: (0, 0))
  half = pl.BlockSpec((1, _RB, _DH), lambda i: (0, i, 0))
  blk = pl.BlockSpec((_RB, _D), lambda i: (i, 0))
  return pl.pallas_call(
      _stage2_body,
      grid=(_N // _RB,),
      in_specs=[half, half, full, vec, full, vec, full, vec, vec, vec,
                full, vec, vec, vec, two],
      out_specs=blk,
      out_shape=jax.ShapeDtypeStruct((_N, _D), jnp.float32),
  )(px2[0:1], px2[1:2], w0, b0, w1, b1, t0w, t0b, g0, be0,
    t1w, t1b, g1, be1, c01)


def kernel(x, hyperedge_index, W0, b0, W1, b1, scale_weights,
           A1_w, A1_b, A2_w, A2_b, T0_w, T0_b, g0, be0, T1_w, T1_b, g1, be1):
  nidx = hyperedge_index[0].astype(jnp.int32)
  eidx = hyperedge_index[1].astype(jnp.int32)

  pad = jnp.full((_EPAD - _E,), _PAD_ID, jnp.int32)
  nid_p = jnp.concatenate([nidx, pad]).reshape(_RTOT, 128)
  eid_p = jnp.concatenate([eidx, pad]).reshape(_RTOT, 128)

  xh = jnp.pad(x.reshape(_N, 2, _DH).transpose(1, 0, 2),
               ((0, 0), (0, _NP - _N), (0, 0))).reshape(2 * _NP, _DH)
  z2 = jnp.zeros((_SL, _DH), jnp.float32)
  z1 = jnp.zeros((_SL,), jnp.float32)

  px2 = _sc_propagate(xh, nid_p, eid_p, z2, z1)

  wd = (A2_w[:, 1] - A2_w[:, 0]).reshape(1, _HID)
  bd = (A2_b[1] - A2_b[0]).reshape(1, 1)
  b0r = b0.reshape(1, _D)
  b1r = b1.reshape(1, _D)

  ssum = _tc_stage1(px2, W0, b0r, W1, b1r, A1_w, A1_b.reshape(1, _HID), wd, bd)
  dyn0 = ssum[0, 0] / _N
  sw = jax.nn.softmax(scale_weights)
  c0 = (sw[0] + dyn0) * 0.5
  c1 = (sw[1] + (1.0 - dyn0)) * 0.5
  c01 = jnp.stack([c0, c1]).reshape(1, 2)

  return _tc_stage2(px2, W0, b0r, W1, b1r,
                    T0_w, T0_b.reshape(1, _D), g0.reshape(1, _D),
                    be0.reshape(1, _D),
                    T1_w, T1_b.reshape(1, _D), g1.reshape(1, _D),
                    be1.reshape(1, _D), c01)
